# baseline (speedup 1.0000x reference)
import jax
import jax.numpy as jnp
from jax.experimental import pallas as pl


def _copy_body(src_ref, dst_ref):
    dst_ref[...] = src_ref[...]


def kernel(x, positions, rel_pos_embeddings):
    B, S, D = x.shape
    xr = x.reshape(B * S, D)
    ROWS = 512
    x_out = pl.pallas_call(
        _copy_body,
        grid=(B * S // ROWS,),
        in_specs=[pl.BlockSpec((ROWS, D), lambda i: (i, 0))],
        out_specs=pl.BlockSpec((ROWS, D), lambda i: (i, 0)),
        out_shape=jax.ShapeDtypeStruct((B * S, D), x.dtype),
    )(xr).reshape(B, S, D)
    pos_out = pl.pallas_call(
        _copy_body,
        out_shape=jax.ShapeDtypeStruct(positions.shape, positions.dtype),
    )(positions)
    return (x_out, pos_out)
